# Initial kernel scaffold; baseline (speedup 1.0000x reference)
import jax
import jax.numpy as jnp
from jax.experimental import pallas as pl


def kernel(x, batch, W1, b1, W2, b2):
    raise NotImplementedError("write your pallas kernel here")


# speedup vs baseline: 4.0021x; 4.0021x over previous
import functools

import jax
import jax.numpy as jnp
from jax import lax
from jax.experimental import pallas as pl
from jax.experimental.pallas import tpu as pltpu
from jax.experimental.pallas import tpu_sc as plsc

_N_NODES = 100000
_HIDDEN = 128
_OUT_SIZE = 10
_NUM_GRAPHS = 512

_NC = 2
_NS = 16
_NW = _NC * _NS
_CHUNK = 80
_ROWS_PER_W = 3200
_CHUNKS_FULL = _ROWS_PER_W // _CHUNK
_CHUNKS_LAST = (_N_NODES - (_NW - 1) * _ROWS_PER_W) // _CHUNK
_CNT_LANES = 128
_GPT = _NUM_GRAPHS // _NS


def _build_seg_pool():
    mesh = plsc.VectorSubcoreMesh(core_axis_name="c", subcore_axis_name="s")

    @functools.partial(
        pl.kernel,
        mesh=mesh,
        out_type=[
            jax.ShapeDtypeStruct((_NC, _NUM_GRAPHS, _HIDDEN), jnp.float32),
            jax.ShapeDtypeStruct((_NC, _NUM_GRAPHS, _CNT_LANES), jnp.float32),
        ],
        scratch_types=[
            pltpu.VMEM((_CHUNK,), jnp.int32),
            pltpu.VMEM((_CHUNK, _HIDDEN), jnp.float32),
            pltpu.VMEM((_CHUNK, _CNT_LANES), jnp.float32),
            pltpu.VMEM((_GPT, _HIDDEN), jnp.float32),
            pltpu.VMEM((_GPT, _CNT_LANES), jnp.float32),
            pltpu.VMEM_SHARED((_NUM_GRAPHS, _HIDDEN), jnp.float32),
            pltpu.VMEM_SHARED((_NUM_GRAPHS, _CNT_LANES), jnp.float32),
        ],
    )
    def seg_pool(x_hbm, batch_hbm, sums_hbm, counts_hbm,
                 idx_v, rows_v, ones_v, zrow_v, zcnt_v, acc_sh, cnt_sh):
        cid = lax.axis_index("c")
        sid = lax.axis_index("s")
        wid = sid * _NC + cid

        zero16 = jnp.zeros((16,), jnp.float32)
        one16 = jnp.full((16,), 1.0, jnp.float32)
        for i in range(_GPT):
            for j in range(_HIDDEN // 16):
                zrow_v[i, pl.ds(j * 16, 16)] = zero16
            for j in range(_CNT_LANES // 16):
                zcnt_v[i, pl.ds(j * 16, 16)] = zero16
        for i in range(_CHUNK):
            for j in range(_CNT_LANES // 16):
                ones_v[i, pl.ds(j * 16, 16)] = one16

        g0 = sid * _GPT
        pltpu.sync_copy(zrow_v, acc_sh.at[pl.ds(g0, _GPT)])
        pltpu.sync_copy(zcnt_v, cnt_sh.at[pl.ds(g0, _GPT)])
        plsc.subcore_barrier()

        base0 = wid * _ROWS_PER_W
        n_chunks = jnp.where(wid == _NW - 1, _CHUNKS_LAST, _CHUNKS_FULL)

        def body(k, carry):
            base = base0 + k * _CHUNK
            pltpu.sync_copy(batch_hbm.at[pl.ds(base, _CHUNK)], idx_v)
            pltpu.sync_copy(x_hbm.at[pl.ds(base, _CHUNK)], rows_v)
            pltpu.sync_copy(rows_v, acc_sh.at[idx_v], add=True)
            pltpu.sync_copy(ones_v, cnt_sh.at[idx_v], add=True)
            return carry

        lax.fori_loop(0, n_chunks, body, 0)
        plsc.subcore_barrier()

        @pl.when(sid == 0)
        def _():
            pltpu.sync_copy(acc_sh, sums_hbm.at[cid])
            pltpu.sync_copy(cnt_sh, counts_hbm.at[cid])

    return seg_pool


_seg_pool = _build_seg_pool()


def _mlp_body(sums_ref, cnts_ref, w1_ref, b1_ref, w2_ref, b2_ref, out_ref):
    sums = sums_ref[0] + sums_ref[1]
    c = jnp.maximum(cnts_ref[0, :, 0:1] + cnts_ref[1, :, 0:1], 1.0)
    pooled = sums / c
    h = jnp.tanh(
        jnp.dot(pooled, w1_ref[...], preferred_element_type=jnp.float32)
        + b1_ref[...]
    )
    out_ref[...] = (
        jnp.dot(h, w2_ref[...], preferred_element_type=jnp.float32)
        + b2_ref[...]
    )


def kernel(x, batch, W1, b1, W2, b2):
    batch_i32 = batch.astype(jnp.int32)
    sums2, counts2 = _seg_pool(x, batch_i32)
    out = pl.pallas_call(
        _mlp_body,
        out_shape=jax.ShapeDtypeStruct((_NUM_GRAPHS, _OUT_SIZE), jnp.float32),
    )(sums2, counts2, W1, b1.reshape(1, -1), W2, b2.reshape(1, -1))
    return out


# speedup vs baseline: 6.4955x; 1.6230x over previous
import functools

import jax
import jax.numpy as jnp
from jax import lax
from jax.experimental import pallas as pl
from jax.experimental.pallas import tpu as pltpu
from jax.experimental.pallas import tpu_sc as plsc

_N_NODES = 100000
_HIDDEN = 128
_OUT_SIZE = 10
_NUM_GRAPHS = 512

_NC = 2
_NS = 16
_NW = _NC * _NS
_CHUNK = 80
_NBUF = 2
_ROWS_PER_W = 3200
_CHUNKS_FULL = _ROWS_PER_W // _CHUNK
_CHUNKS_LAST = (_N_NODES - (_NW - 1) * _ROWS_PER_W) // _CHUNK
_CNT_LANES = 128
_GPT = _NUM_GRAPHS // _NS


def _build_seg_pool():
    mesh = plsc.VectorSubcoreMesh(core_axis_name="c", subcore_axis_name="s")

    @functools.partial(
        pl.kernel,
        mesh=mesh,
        out_type=[
            jax.ShapeDtypeStruct((_NC, _NUM_GRAPHS, _HIDDEN), jnp.float32),
            jax.ShapeDtypeStruct((_NC, _NUM_GRAPHS, _CNT_LANES), jnp.float32),
        ],
        scratch_types=(
            [pltpu.VMEM((_CHUNK,), jnp.int32) for _ in range(_NBUF)]
            + [pltpu.VMEM((_CHUNK, _HIDDEN), jnp.float32) for _ in range(_NBUF)]
            + [
                pltpu.VMEM((_CHUNK, _CNT_LANES), jnp.float32),
                pltpu.VMEM((_GPT, _HIDDEN), jnp.float32),
                pltpu.VMEM((_GPT, _CNT_LANES), jnp.float32),
                pltpu.VMEM_SHARED((_NUM_GRAPHS, _HIDDEN), jnp.float32),
                pltpu.VMEM_SHARED((_NUM_GRAPHS, _CNT_LANES), jnp.float32),
            ]
            + [pltpu.SemaphoreType.DMA for _ in range(_NBUF)]
        ),
    )
    def seg_pool(x_hbm, batch_hbm, sums_hbm, counts_hbm, *refs):
        idx_v = refs[0:_NBUF]
        rows_v = refs[_NBUF:2 * _NBUF]
        ones_v, zrow_v, zcnt_v, acc_sh, cnt_sh = refs[2 * _NBUF:2 * _NBUF + 5]
        gsem = refs[2 * _NBUF + 5:]
        cid = lax.axis_index("c")
        sid = lax.axis_index("s")
        wid = sid * _NC + cid

        zero16 = jnp.zeros((16,), jnp.float32)
        one16 = jnp.full((16,), 1.0, jnp.float32)
        for i in range(_GPT):
            for j in range(_HIDDEN // 16):
                zrow_v[i, pl.ds(j * 16, 16)] = zero16
            for j in range(_CNT_LANES // 16):
                zcnt_v[i, pl.ds(j * 16, 16)] = zero16
        for i in range(_CHUNK):
            for j in range(_CNT_LANES // 16):
                ones_v[i, pl.ds(j * 16, 16)] = one16

        base0 = wid * _ROWS_PER_W
        n_chunks = jnp.where(wid == _NW - 1, _CHUNKS_LAST, _CHUNKS_FULL)

        def g_start(b, k):
            base = base0 + k * _CHUNK
            pltpu.async_copy(batch_hbm.at[pl.ds(base, _CHUNK)], idx_v[b],
                             gsem[b])
            pltpu.async_copy(x_hbm.at[pl.ds(base, _CHUNK)], rows_v[b],
                             gsem[b])

        def g_wait(b, k):
            base = base0 + k * _CHUNK
            pltpu.make_async_copy(batch_hbm.at[pl.ds(base, _CHUNK)], idx_v[b],
                                  gsem[b]).wait()
            pltpu.make_async_copy(x_hbm.at[pl.ds(base, _CHUNK)], rows_v[b],
                                  gsem[b]).wait()

        g0 = sid * _GPT
        pltpu.sync_copy(zrow_v, acc_sh.at[pl.ds(g0, _GPT)])
        pltpu.sync_copy(zcnt_v, cnt_sh.at[pl.ds(g0, _GPT)])
        plsc.subcore_barrier()

        g_start(0, 0)
        g_start(1, 1)

        def outer(o, carry):
            for j in range(_NBUF):
                k = o * _NBUF + j

                @pl.when(k < n_chunks)
                def _(j=j, k=k):
                    g_wait(j, k)
                    pltpu.sync_copy(rows_v[j], acc_sh.at[idx_v[j]], add=True)
                    pltpu.sync_copy(ones_v, cnt_sh.at[idx_v[j]], add=True)

                @pl.when(k + _NBUF < n_chunks)
                def _(j=j, k=k):
                    g_start(j, k + _NBUF)
            return carry

        lax.fori_loop(0, _CHUNKS_FULL // _NBUF, outer, 0)
        plsc.subcore_barrier()

        @pl.when(sid == 0)
        def _():
            pltpu.sync_copy(acc_sh, sums_hbm.at[cid])
            pltpu.sync_copy(cnt_sh, counts_hbm.at[cid])

    return seg_pool


_seg_pool = _build_seg_pool()


def _mlp_body(sums_ref, cnts_ref, w1_ref, b1_ref, w2_ref, b2_ref, out_ref):
    sums = sums_ref[0] + sums_ref[1]
    c = jnp.maximum(cnts_ref[0, :, 0:1] + cnts_ref[1, :, 0:1], 1.0)
    pooled = sums / c
    h = jnp.tanh(
        jnp.dot(pooled, w1_ref[...], preferred_element_type=jnp.float32)
        + b1_ref[...]
    )
    out_ref[...] = (
        jnp.dot(h, w2_ref[...], preferred_element_type=jnp.float32)
        + b2_ref[...]
    )


def kernel(x, batch, W1, b1, W2, b2):
    batch_i32 = batch.astype(jnp.int32)
    sums2, counts2 = _seg_pool(x, batch_i32)
    out = pl.pallas_call(
        _mlp_body,
        out_shape=jax.ShapeDtypeStruct((_NUM_GRAPHS, _OUT_SIZE), jnp.float32),
    )(sums2, counts2, W1, b1.reshape(1, -1), W2, b2.reshape(1, -1))
    return out


# speedup vs baseline: 7.3960x; 1.1386x over previous
import functools

import jax
import jax.numpy as jnp
from jax import lax
from jax.experimental import pallas as pl
from jax.experimental.pallas import tpu as pltpu
from jax.experimental.pallas import tpu_sc as plsc

_N_NODES = 100000
_HIDDEN = 128
_OUT_SIZE = 10
_NUM_GRAPHS = 512

_NC = 2
_NS = 16
_NW = _NC * _NS
_CHUNK = 80
_NBUF = 2
_ROWS_PER_W = 3200
_CHUNKS_FULL = _ROWS_PER_W // _CHUNK
_CHUNKS_LAST = (_N_NODES - (_NW - 1) * _ROWS_PER_W) // _CHUNK
_CNT_ROWS = 16
_GPT = _NUM_GRAPHS // _NS


def _build_seg_pool():
    mesh = plsc.VectorSubcoreMesh(core_axis_name="c", subcore_axis_name="s")

    @functools.partial(
        pl.kernel,
        mesh=mesh,
        out_type=[
            jax.ShapeDtypeStruct((_NC, _NUM_GRAPHS, _HIDDEN), jnp.float32),
            jax.ShapeDtypeStruct((_NC, _CNT_ROWS, _HIDDEN), jnp.float32),
        ],
        scratch_types=(
            [pltpu.VMEM((_CHUNK,), jnp.int32) for _ in range(_NBUF)]
            + [pltpu.VMEM((_CHUNK, _HIDDEN), jnp.float32)
               for _ in range(_NBUF)]
            + [
                pltpu.VMEM((_GPT, _HIDDEN), jnp.float32),
                pltpu.VMEM((_NUM_GRAPHS,), jnp.float32),
                pltpu.VMEM((_CNT_ROWS, _HIDDEN), jnp.float32),
                pltpu.VMEM((_CNT_ROWS,), jnp.int32),
                pltpu.SMEM((_NUM_GRAPHS,), jnp.float32),
                pltpu.VMEM_SHARED((_NUM_GRAPHS, _HIDDEN), jnp.float32),
                pltpu.VMEM_SHARED((_CNT_ROWS, _HIDDEN), jnp.float32),
            ]
            + [pltpu.SemaphoreType.DMA for _ in range(_NBUF + 1)]
        ),
    )
    def seg_pool(x_hbm, batch_hbm, iota_hbm, sums_hbm, counts_hbm, *refs):
        idx_v = refs[0:_NBUF]
        rows_v = refs[_NBUF:2 * _NBUF]
        (zrow_v, cflat_v, c2d_v, idx4_v, hist_s, acc_sh, cnt_sh) = \
            refs[2 * _NBUF:2 * _NBUF + 7]
        gsem = refs[2 * _NBUF + 7:2 * _NBUF + 7 + _NBUF]
        ssem = refs[2 * _NBUF + 7 + _NBUF]
        cid = lax.axis_index("c")
        sid = lax.axis_index("s")
        wid = sid * _NC + cid

        zero16 = jnp.zeros((16,), jnp.float32)
        for i in range(_GPT):
            for j in range(_HIDDEN // 16):
                zrow_v[i, pl.ds(j * 16, 16)] = zero16
        for i in range(_CNT_ROWS):
            for j in range(_HIDDEN // 16):
                c2d_v[i, pl.ds(j * 16, 16)] = zero16

        def zero_hist(i, carry):
            hist_s[i] = 0.0
            return carry

        lax.fori_loop(0, _NUM_GRAPHS, zero_hist, 0)
        pltpu.sync_copy(iota_hbm, idx4_v)

        g0 = sid * _GPT
        pltpu.sync_copy(zrow_v, acc_sh.at[pl.ds(g0, _GPT)])

        @pl.when(sid == 0)
        def _():
            pltpu.sync_copy(c2d_v, cnt_sh)

        plsc.subcore_barrier()

        base0 = wid * _ROWS_PER_W
        n_chunks = jnp.where(wid == _NW - 1, _CHUNKS_LAST, _CHUNKS_FULL)

        def g_start(b, k):
            base = base0 + k * _CHUNK
            pltpu.async_copy(batch_hbm.at[pl.ds(base, _CHUNK)], idx_v[b],
                             gsem[b])
            pltpu.async_copy(x_hbm.at[pl.ds(base, _CHUNK)], rows_v[b],
                             gsem[b])

        def g_wait(b, k):
            base = base0 + k * _CHUNK
            pltpu.make_async_copy(batch_hbm.at[pl.ds(base, _CHUNK)], idx_v[b],
                                  gsem[b]).wait()
            pltpu.make_async_copy(x_hbm.at[pl.ds(base, _CHUNK)], rows_v[b],
                                  gsem[b]).wait()

        g_start(0, 0)
        g_start(1, 1)

        def outer(o, carry):
            for j in range(_NBUF):
                k = o * _NBUF + j

                @pl.when(k < n_chunks)
                def _(j=j, k=k):
                    g_wait(j, k)
                    sc = pltpu.async_copy(rows_v[j], acc_sh.at[idx_v[j]],
                                          ssem, add=True)

                    def hist_body(v, carry, j=j):
                        ids16 = idx_v[j][pl.ds(v * 16, 16)]
                        for l in range(16):
                            g = ids16[l]
                            hist_s[g] = hist_s[g] + 1.0
                        return carry

                    lax.fori_loop(0, _CHUNK // 16, hist_body, 0)
                    sc.wait()

                @pl.when(k + _NBUF < n_chunks)
                def _(j=j, k=k):
                    g_start(j, k + _NBUF)
            return carry

        lax.fori_loop(0, _CHUNKS_FULL // _NBUF, outer, 0)

        lane = lax.iota(jnp.int32, 16)

        def build_body(r, carry):
            v = jnp.zeros((16,), jnp.float32)
            for l in range(16):
                s = hist_s[r * 16 + l]
                v = jnp.where(lane == l, s, v)
            cflat_v[pl.ds(r * 16, 16)] = v
            return carry

        lax.fori_loop(0, _NUM_GRAPHS // 16, build_body, 0)
        for i in range(_NUM_GRAPHS // _HIDDEN):
            for j in range(_HIDDEN // 16):
                c2d_v[i, pl.ds(j * 16, 16)] = \
                    cflat_v[pl.ds(i * _HIDDEN + j * 16, 16)]
        pltpu.sync_copy(c2d_v, cnt_sh.at[idx4_v], add=True)
        plsc.subcore_barrier()

        @pl.when(sid == 0)
        def _():
            pltpu.sync_copy(acc_sh, sums_hbm.at[cid])
            pltpu.sync_copy(cnt_sh, counts_hbm.at[cid])

    return seg_pool


_seg_pool = _build_seg_pool()


def _mlp_body(sums_ref, cnts_ref, w1_ref, b1_ref, w2_ref, b2_ref, out_ref):
    sums = sums_ref[0] + sums_ref[1]
    c = jnp.maximum(cnts_ref[0] + cnts_ref[1], 1.0)
    pooled = sums / c
    h = jnp.tanh(
        jnp.dot(pooled, w1_ref[...], preferred_element_type=jnp.float32)
        + b1_ref[...]
    )
    out_ref[...] = (
        jnp.dot(h, w2_ref[...], preferred_element_type=jnp.float32)
        + b2_ref[...]
    )


def kernel(x, batch, W1, b1, W2, b2):
    batch_i32 = batch.astype(jnp.int32)
    iota16 = jnp.arange(_CNT_ROWS, dtype=jnp.int32)
    sums2, counts2 = _seg_pool(x, batch_i32, iota16)
    counts_col = counts2[:, : _NUM_GRAPHS // _HIDDEN, :].reshape(
        _NC, _NUM_GRAPHS, 1)
    out = pl.pallas_call(
        _mlp_body,
        out_shape=jax.ShapeDtypeStruct((_NUM_GRAPHS, _OUT_SIZE), jnp.float32),
    )(sums2, counts_col, W1, b1.reshape(1, -1), W2, b2.reshape(1, -1))
    return out


# speedup vs baseline: 7.4366x; 1.0055x over previous
import functools

import jax
import jax.numpy as jnp
from jax import lax
from jax.experimental import pallas as pl
from jax.experimental.pallas import tpu as pltpu
from jax.experimental.pallas import tpu_sc as plsc

_N_NODES = 100000
_HIDDEN = 128
_OUT_SIZE = 10
_NUM_GRAPHS = 512

_NC = 2
_NS = 16
_NW = _NC * _NS
_CHUNK = 128
_NBUF = 2
_CHUNKS_A = 25
_CHUNKS_B = 24
_NW_A = 13
_TAIL_BASE = 781 * _CHUNK
_TAIL = _N_NODES - _TAIL_BASE
_CNT_ROWS = 16
_GPT = _NUM_GRAPHS // _NS


def _build_seg_pool():
    mesh = plsc.VectorSubcoreMesh(core_axis_name="c", subcore_axis_name="s")

    @functools.partial(
        pl.kernel,
        mesh=mesh,
        out_type=[
            jax.ShapeDtypeStruct((_NC, _NUM_GRAPHS, _HIDDEN), jnp.float32),
            jax.ShapeDtypeStruct((_NC, _CNT_ROWS, _HIDDEN), jnp.float32),
        ],
        scratch_types=(
            [pltpu.VMEM((_CHUNK,), jnp.int32) for _ in range(_NBUF)]
            + [pltpu.VMEM((_CHUNK, _HIDDEN), jnp.float32)
               for _ in range(_NBUF)]
            + [
                pltpu.VMEM((_GPT, _HIDDEN), jnp.float32),
                pltpu.VMEM((_NUM_GRAPHS,), jnp.float32),
                pltpu.VMEM((_CNT_ROWS, _HIDDEN), jnp.float32),
                pltpu.VMEM((_CNT_ROWS,), jnp.int32),
                pltpu.VMEM((_TAIL,), jnp.int32),
                pltpu.VMEM((_TAIL, _HIDDEN), jnp.float32),
                pltpu.SMEM((_NUM_GRAPHS,), jnp.float32),
                pltpu.VMEM_SHARED((_NUM_GRAPHS, _HIDDEN), jnp.float32),
                pltpu.VMEM_SHARED((_CNT_ROWS, _HIDDEN), jnp.float32),
            ]
            + [pltpu.SemaphoreType.DMA for _ in range(_NBUF + 1)]
        ),
    )
    def seg_pool(x_hbm, batch_hbm, iota_hbm, sums_hbm, counts_hbm, *refs):
        idx_v = refs[0:_NBUF]
        rows_v = refs[_NBUF:2 * _NBUF]
        (zrow_v, cflat_v, c2d_v, idx4_v, idxt_v, rowst_v, hist_s,
         acc_sh, cnt_sh) = refs[2 * _NBUF:2 * _NBUF + 9]
        gsem = refs[2 * _NBUF + 9:2 * _NBUF + 9 + _NBUF]
        ssem = refs[2 * _NBUF + 9 + _NBUF]
        cid = lax.axis_index("c")
        sid = lax.axis_index("s")
        wid = sid * _NC + cid

        zero16 = jnp.zeros((16,), jnp.float32)
        for i in range(_GPT):
            for j in range(_HIDDEN // 16):
                zrow_v[i, pl.ds(j * 16, 16)] = zero16
        for i in range(_CNT_ROWS):
            for j in range(_HIDDEN // 16):
                c2d_v[i, pl.ds(j * 16, 16)] = zero16

        def zero_hist(i, carry):
            hist_s[i] = 0.0
            return carry

        lax.fori_loop(0, _NUM_GRAPHS, zero_hist, 0)
        pltpu.sync_copy(iota_hbm, idx4_v)

        g0 = sid * _GPT
        pltpu.sync_copy(zrow_v, acc_sh.at[pl.ds(g0, _GPT)])

        @pl.when(sid == 0)
        def _():
            pltpu.sync_copy(c2d_v, cnt_sh)

        plsc.subcore_barrier()

        base0 = jnp.where(
            wid < _NW_A,
            wid * _CHUNKS_A * _CHUNK,
            _NW_A * _CHUNKS_A * _CHUNK + (wid - _NW_A) * _CHUNKS_B * _CHUNK,
        )
        n_chunks = jnp.where(wid < _NW_A, _CHUNKS_A, _CHUNKS_B)

        def g_start(b, k):
            base = base0 + k * _CHUNK
            pltpu.async_copy(batch_hbm.at[pl.ds(base, _CHUNK)], idx_v[b],
                             gsem[b])
            pltpu.async_copy(x_hbm.at[pl.ds(base, _CHUNK)], rows_v[b],
                             gsem[b])

        def g_wait(b, k):
            base = base0 + k * _CHUNK
            pltpu.make_async_copy(batch_hbm.at[pl.ds(base, _CHUNK)], idx_v[b],
                                  gsem[b]).wait()
            pltpu.make_async_copy(x_hbm.at[pl.ds(base, _CHUNK)], rows_v[b],
                                  gsem[b]).wait()

        g_start(0, 0)
        g_start(1, 1)

        def outer(o, carry):
            for j in range(_NBUF):
                k = o * _NBUF + j

                @pl.when(k < n_chunks)
                def _(j=j, k=k):
                    g_wait(j, k)
                    sc = pltpu.async_copy(rows_v[j], acc_sh.at[idx_v[j]],
                                          ssem, add=True)

                    def hist_body(v, carry, j=j):
                        ids16 = idx_v[j][pl.ds(v * 16, 16)]
                        for l in range(16):
                            g = ids16[l]
                            hist_s[g] = hist_s[g] + 1.0
                        return carry

                    lax.fori_loop(0, _CHUNK // 16, hist_body, 0)
                    sc.wait()

                @pl.when(k + _NBUF < n_chunks)
                def _(j=j, k=k):
                    g_start(j, k + _NBUF)
            return carry

        lax.fori_loop(0, (_CHUNKS_A + _NBUF - 1) // _NBUF, outer, 0)

        @pl.when(wid == _NW - 1)
        def _():
            pltpu.sync_copy(batch_hbm.at[pl.ds(_TAIL_BASE, _TAIL)], idxt_v)
            pltpu.sync_copy(x_hbm.at[pl.ds(_TAIL_BASE, _TAIL)], rowst_v)
            sc = pltpu.async_copy(rowst_v, acc_sh.at[idxt_v], ssem, add=True)
            for v in range(_TAIL // 16):
                ids16 = idxt_v[pl.ds(v * 16, 16)]
                for l in range(16):
                    g = ids16[l]
                    hist_s[g] = hist_s[g] + 1.0
            sc.wait()

        lane = lax.iota(jnp.int32, 16)

        def build_body(r, carry):
            v = jnp.zeros((16,), jnp.float32)
            for l in range(16):
                s = hist_s[r * 16 + l]
                v = jnp.where(lane == l, s, v)
            cflat_v[pl.ds(r * 16, 16)] = v
            return carry

        lax.fori_loop(0, _NUM_GRAPHS // 16, build_body, 0)
        for i in range(_NUM_GRAPHS // _HIDDEN):
            for j in range(_HIDDEN // 16):
                c2d_v[i, pl.ds(j * 16, 16)] = \
                    cflat_v[pl.ds(i * _HIDDEN + j * 16, 16)]
        pltpu.sync_copy(c2d_v, cnt_sh.at[idx4_v], add=True)
        plsc.subcore_barrier()

        @pl.when(sid == 0)
        def _():
            pltpu.sync_copy(acc_sh, sums_hbm.at[cid])
            pltpu.sync_copy(cnt_sh, counts_hbm.at[cid])

    return seg_pool


_seg_pool = _build_seg_pool()


def _mlp_body(sums_ref, cnts_ref, w1_ref, b1_ref, w2_ref, b2_ref, out_ref):
    sums = sums_ref[0] + sums_ref[1]
    c = jnp.maximum(cnts_ref[0] + cnts_ref[1], 1.0)
    pooled = sums / c
    h = jnp.tanh(
        jnp.dot(pooled, w1_ref[...], preferred_element_type=jnp.float32)
        + b1_ref[...]
    )
    out_ref[...] = (
        jnp.dot(h, w2_ref[...], preferred_element_type=jnp.float32)
        + b2_ref[...]
    )


def kernel(x, batch, W1, b1, W2, b2):
    batch_i32 = batch.astype(jnp.int32)
    iota16 = jnp.arange(_CNT_ROWS, dtype=jnp.int32)
    sums2, counts2 = _seg_pool(x, batch_i32, iota16)
    counts_col = counts2[:, : _NUM_GRAPHS // _HIDDEN, :].reshape(
        _NC, _NUM_GRAPHS, 1)
    out = pl.pallas_call(
        _mlp_body,
        out_shape=jax.ShapeDtypeStruct((_NUM_GRAPHS, _OUT_SIZE), jnp.float32),
    )(sums2, counts_col, W1, b1.reshape(1, -1), W2, b2.reshape(1, -1))
    return out
